# baseline (speedup 1.0000x reference)
import functools

import jax
import jax.numpy as jnp
from jax import lax
from jax.experimental import pallas as pl
from jax.experimental.pallas import tpu as pltpu
from jax.experimental.pallas import tpu_sc as plsc

_CHUNK = 128


@functools.partial(jax.jit, static_argnames=())
def _sc_gather(table, idx3):
    NW, n_chunks, C = idx3.shape
    D = table.shape[1]
    N = NW * n_chunks * C
    info = plsc.get_sparse_core_info()
    num_cores = info.num_cores
    mesh = plsc.VectorSubcoreMesh(core_axis_name="c", subcore_axis_name="s")

    @functools.partial(
        pl.kernel,
        mesh=mesh,
        out_type=jax.ShapeDtypeStruct((N, D), jnp.float32),
        scratch_types=[
            pltpu.VMEM((n_chunks, C), jnp.int32),
            pltpu.VMEM((C, D), jnp.float32),
            pltpu.SemaphoreType.DMA,
        ],
        compiler_params=pltpu.CompilerParams(use_tc_tiling_on_sc=False),
    )
    def k(table_hbm, idx_hbm, out_hbm, idx_v, rows_v, sem):
        wid = lax.axis_index("s") * num_cores + lax.axis_index("c")
        pltpu.sync_copy(idx_hbm.at[wid], idx_v)
        base = wid * (n_chunks * C)

        def body(j, carry):
            pltpu.async_copy(table_hbm.at[idx_v.at[j]], rows_v, sem).wait()
            pltpu.sync_copy(rows_v, out_hbm.at[pl.ds(base + j * C, C)])
            return carry

        lax.fori_loop(0, n_chunks, body, 0)

    return k(table, idx3)


def _tc_times(row1, tp, fc_w, fc_b):
    B, H = row1.shape
    D = fc_w.shape[1]
    BB = 256

    def body(r1_ref, tp_ref, w_ref, b_ref, o_ref):
        gap = tp_ref[...] - r1_ref[...]
        t = jnp.log(gap + 1.0)
        w = jnp.reshape(w_ref[...], (1, 1, D))
        b = jnp.reshape(b_ref[...], (1, 1, D))
        o_ref[...] = t[:, :, None] * w + b

    return pl.pallas_call(
        body,
        grid=(B // BB,),
        in_specs=[
            pl.BlockSpec((BB, H), lambda i: (i, 0)),
            pl.BlockSpec((BB, 1), lambda i: (i, 0)),
            pl.BlockSpec((1, D), lambda i: (0, 0)),
            pl.BlockSpec((1, D), lambda i: (0, 0)),
        ],
        out_specs=pl.BlockSpec((BB, H, D), lambda i: (i, 0, 0)),
        out_shape=jax.ShapeDtypeStruct((B, H, D), jnp.float32),
    )(row1, tp, fc_w, fc_b)


def kernel(row0, row1, row2, row3, row4, row5, row6, row7,
           click_emb, exposure_emb, uid_emb, fc_w, fc_b):
    B, H = row0.shape
    D = click_emb.shape[1]
    info = plsc.get_sparse_core_info()
    NW = info.num_cores * info.num_subcores
    n_chunks = (B * H) // (NW * _CHUNK)
    idx3 = row0.astype(jnp.int32).reshape(NW, n_chunks, _CHUNK)
    high_items_emb = _sc_gather(click_emb, idx3).reshape(B, H, D)
    tp = row6[:, -1][:, None]
    fc_b2 = jnp.reshape(fc_b, (1, D))
    high_times_emb = _tc_times(row1, tp, fc_w, fc_b2)
    sep_emb = exposure_emb[0]
    return (high_items_emb, high_times_emb, sep_emb)
